# baseline (speedup 1.0000x reference)
import jax
import jax.numpy as jnp
from jax import lax
from jax.experimental import pallas as pl
from jax.experimental.pallas import tpu as pltpu
from jax.experimental.pallas import tpu_sc as plsc

_NUM_EMB = 1000000
_D = 64
_B = 4096
_H = 50

_INFO = plsc.get_sparse_core_info()
_NC = _INFO.num_cores
_NS = _INFO.num_subcores
_NW = _NC * _NS
_TOTAL = _B * _H
_CH = 128
_NCH_TOT = _TOTAL // _CH
_NCH = _NCH_TOT // _NW
_NBUF = 5
_BC = _B // _CH
_C = 16384
_KSH = 30 * _C
_N2 = 32 * _C


def _tc_prep_body(a_ref, b_ref, out_ref):
  out_ref[...] = jnp.concatenate([a_ref[...].T, b_ref[...].T], axis=1)


def _tc_prep(tt):
  return pl.pallas_call(
      _tc_prep_body,
      grid=(_N2 // _C,),
      in_specs=[
          pl.BlockSpec((_D, _C), lambda i: (0, i)),
          pl.BlockSpec((_D, _C), lambda i: (0, i + _KSH // _C)),
      ],
      out_specs=pl.BlockSpec((_C, 2 * _D), lambda i: (i, 0)),
      out_shape=jax.ShapeDtypeStruct((_N2, 2 * _D), jnp.float32),
  )(tt, tt)


def _sc_body(idx_hbm, t2_hbm, out_hbm, idx2_v, *rest):
  gbufs = rest[:_NBUF]
  gsems = rest[_NBUF:2 * _NBUF]
  wsems = rest[2 * _NBUF:3 * _NBUF]

  wid = lax.axis_index("s") * _NC + lax.axis_index("c")
  k0 = wid * _NCH
  p0 = k0 // 8
  loc = k0 - p0 * 8

  pltpu.sync_copy(idx_hbm.at[pl.ds(p0, 7)], idx2_v)

  @pl.loop(0, 56)
  def _(t):
    r1 = t // 8
    r2 = t - r1 * 8
    for g in range(8):
      v = idx2_v[r1, r2, pl.ds(g * 16, 16)]
      hi = (v >= _KSH).astype(jnp.int32)
      idx2_v[r1, r2, pl.ds(g * 16, 16)] = v - hi * _KSH

  def rowsel(c):
    t = loc + c
    r1 = t // 8
    return r1, t - r1 * 8

  def start_gather(c, b):
    r1, r2 = rowsel(c)
    pltpu.async_copy(t2_hbm.at[idx2_v.at[r1, r2]], gbufs[b], gsems[b])

  def wait_gather(c, b):
    r1, r2 = rowsel(c)
    pltpu.make_async_copy(
        t2_hbm.at[idx2_v.at[r1, r2]], gbufs[b], gsems[b]).wait()

  def start_write(c, b):
    pltpu.async_copy(
        gbufs[b], out_hbm.at[pl.ds((k0 + c) * _CH, _CH)], wsems[b])

  def wait_write(c, b):
    pltpu.make_async_copy(
        gbufs[b], out_hbm.at[pl.ds((k0 + c) * _CH, _CH)], wsems[b]).wait()

  for b in range(_NBUF):
    start_gather(b, b)

  @pl.loop(0, _NCH - _NBUF, step=_NBUF)
  def _(i0):
    for b in range(_NBUF):
      i = i0 + b
      wait_gather(i, b)
      start_write(i, b)
      wait_write(i, b)
      start_gather(i + _NBUF, b)

  for b in range(_NBUF):
    i = _NCH - _NBUF + b
    wait_gather(i, b)
    start_write(i, b)
  for b in range(_NBUF):
    i = _NCH - _NBUF + b
    wait_write(i, b)


@jax.jit
def _run(idx, t2):
  mesh = plsc.VectorSubcoreMesh(core_axis_name="c", subcore_axis_name="s")
  scratch = (
      [pltpu.VMEM((7, 8, _CH), jnp.int32)]
      + [pltpu.VMEM((_CH, 2 * _D), jnp.float32) for _ in range(_NBUF)]
      + [pltpu.SemaphoreType.DMA for _ in range(2 * _NBUF)]
  )
  return pl.kernel(
      _sc_body,
      out_type=jax.ShapeDtypeStruct((_TOTAL, 2 * _D), jnp.float32),
      mesh=mesh,
      scratch_types=scratch,
      compiler_params=pltpu.CompilerParams(
          use_tc_tiling_on_sc=True, needs_layout_passes=False),
  )(idx, t2)


_SB = _B // 2


def _tc_sel_body(raw_ref, hi_ref, out_ref):
  raw = raw_ref[...]
  low = raw[:, :_D].T
  high = raw[:, _D:].T
  mask = jnp.broadcast_to(hi_ref[...].reshape(1, _SB) != 0, (_D, _SB))
  out_ref[...] = jnp.where(mask, high, low).reshape(1, _D, _SB)


def _tc_select(raw, hi):
  nc = _B // _SB
  return pl.pallas_call(
      _tc_sel_body,
      grid=(_H, nc),
      in_specs=[
          pl.BlockSpec((_SB, 2 * _D), lambda h, c: (h * nc + c, 0)),
          pl.BlockSpec((1, 1, _SB), lambda h, c: (h * nc + c, 0, 0)),
      ],
      out_specs=pl.BlockSpec((1, _D, _SB), lambda h, c: (h, 0, c)),
      out_shape=jax.ShapeDtypeStruct((_H, _D, _B), jnp.float32),
  )(raw, hi)


def kernel(indices, table):
  idx_t = indices.T.astype(jnp.int32)
  idx = idx_t.reshape(_NCH_TOT // 8, 8, _CH)
  hi = (idx_t >= _KSH).astype(jnp.int32).reshape(_TOTAL // _SB, 1, _SB)
  t2 = _tc_prep(table.T)
  raw = _run(idx, t2)
  out = _tc_select(raw, hi)
  return out.transpose(2, 0, 1)


# speedup vs baseline: 1.0641x; 1.0641x over previous
import jax
import jax.numpy as jnp
from jax import lax
from jax.experimental import pallas as pl
from jax.experimental.pallas import tpu as pltpu
from jax.experimental.pallas import tpu_sc as plsc

_NUM_EMB = 1000000
_D = 64
_B = 4096
_H = 50

_INFO = plsc.get_sparse_core_info()
_NC = _INFO.num_cores
_NS = _INFO.num_subcores
_NW = _NC * _NS
_TOTAL = _B * _H
_CH = 128
_NCH_TOT = _TOTAL // _CH
_NCH = _NCH_TOT // _NW
_NBUF = 5
_BC = _B // _CH
_C = 16384
_KSH = 30 * _C
_N2 = 32 * _C


def _tc_prep_body(a_ref, b_ref, out_ref):
  out_ref[...] = jnp.concatenate([a_ref[...].T, b_ref[...].T], axis=1)


def _tc_prep(tt):
  return pl.pallas_call(
      _tc_prep_body,
      grid=(_N2 // _C,),
      in_specs=[
          pl.BlockSpec((_D, _C), lambda i: (0, i)),
          pl.BlockSpec((_D, _C), lambda i: (0, i + _KSH // _C)),
      ],
      out_specs=pl.BlockSpec((_C, 2 * _D), lambda i: (i, 0)),
      out_shape=jax.ShapeDtypeStruct((_N2, 2 * _D), jnp.float32),
  )(tt, tt)


def _sc_body(idx_hbm, t2_hbm, out_hbm, idx2_v, *rest):
  gbufs = rest[:_NBUF]
  gsems = rest[_NBUF:2 * _NBUF]
  wsems = rest[2 * _NBUF:3 * _NBUF]

  wid = lax.axis_index("s") * _NC + lax.axis_index("c")
  k0 = wid * _NCH
  p0 = k0 // 8
  loc = k0 - p0 * 8

  pltpu.sync_copy(idx_hbm.at[pl.ds(p0, 7)], idx2_v)

  @pl.loop(0, 56)
  def _(t):
    r1 = t // 8
    r2 = t - r1 * 8
    for g in range(8):
      v = idx2_v[r1, r2, pl.ds(g * 16, 16)]
      hi = (v >= _KSH).astype(jnp.int32)
      idx2_v[r1, r2, pl.ds(g * 16, 16)] = v - hi * _KSH

  def rowsel(c):
    t = loc + c
    r1 = t // 8
    return r1, t - r1 * 8

  def start_gather(c, b):
    r1, r2 = rowsel(c)
    pltpu.async_copy(t2_hbm.at[idx2_v.at[r1, r2]], gbufs[b], gsems[b])

  def wait_gather(c, b):
    r1, r2 = rowsel(c)
    pltpu.make_async_copy(
        t2_hbm.at[idx2_v.at[r1, r2]], gbufs[b], gsems[b]).wait()

  def start_write(c, b):
    pltpu.async_copy(
        gbufs[b], out_hbm.at[pl.ds((k0 + c) * _CH, _CH)], wsems[b])

  def wait_write(c, b):
    pltpu.make_async_copy(
        gbufs[b], out_hbm.at[pl.ds((k0 + c) * _CH, _CH)], wsems[b]).wait()

  for b in range(_NBUF):
    start_gather(b, b)

  @pl.loop(0, _NCH - _NBUF, step=_NBUF)
  def _(i0):
    for b in range(_NBUF):
      i = i0 + b
      wait_gather(i, b)
      start_write(i, b)
      wait_write(i, b)
      start_gather(i + _NBUF, b)

  for b in range(_NBUF):
    i = _NCH - _NBUF + b
    wait_gather(i, b)
    start_write(i, b)
  for b in range(_NBUF):
    i = _NCH - _NBUF + b
    wait_write(i, b)


@jax.jit
def _run(idx, t2):
  mesh = plsc.VectorSubcoreMesh(core_axis_name="c", subcore_axis_name="s")
  scratch = (
      [pltpu.VMEM((7, 8, _CH), jnp.int32)]
      + [pltpu.VMEM((_CH, 2 * _D), jnp.float32) for _ in range(_NBUF)]
      + [pltpu.SemaphoreType.DMA for _ in range(2 * _NBUF)]
  )
  return pl.kernel(
      _sc_body,
      out_type=jax.ShapeDtypeStruct((_TOTAL, 2 * _D), jnp.float32),
      mesh=mesh,
      scratch_types=scratch,
      compiler_params=pltpu.CompilerParams(
          use_tc_tiling_on_sc=True, needs_layout_passes=False),
  )(idx, t2)


_SB = _B


def _tc_sel_body(raw_ref, hi_ref, out_ref):
  raw = raw_ref[...]
  low = raw[:, :_D].T
  high = raw[:, _D:].T
  mask = jnp.broadcast_to(hi_ref[...].reshape(1, _SB) != 0, (_D, _SB))
  out_ref[...] = jnp.where(mask, high, low).reshape(1, _D, _SB)


def _tc_select(raw, hi):
  nc = _B // _SB
  return pl.pallas_call(
      _tc_sel_body,
      grid=(_H, nc),
      in_specs=[
          pl.BlockSpec((_SB, 2 * _D), lambda h, c: (h * nc + c, 0)),
          pl.BlockSpec((1, 1, _SB), lambda h, c: (h * nc + c, 0, 0)),
      ],
      out_specs=pl.BlockSpec((1, _D, _SB), lambda h, c: (h, 0, c)),
      out_shape=jax.ShapeDtypeStruct((_H, _D, _B), jnp.float32),
  )(raw, hi)


def kernel(indices, table):
  idx_t = indices.T.astype(jnp.int32)
  idx = idx_t.reshape(_NCH_TOT // 8, 8, _CH)
  hi = (idx_t >= _KSH).astype(jnp.int32).reshape(_TOTAL // _SB, 1, _SB)
  t2 = _tc_prep(table.T)
  raw = _run(idx, t2)
  out = _tc_select(raw, hi)
  return out.transpose(2, 0, 1)
